# Initial kernel scaffold; baseline (speedup 1.0000x reference)
import jax
import jax.numpy as jnp
from jax.experimental import pallas as pl


def kernel(token_ids, W_emb, W1, b1, W2, b2):
    raise NotImplementedError("write your pallas kernel here")


# speedup vs baseline: 14.5863x; 14.5863x over previous
import functools

import jax
import jax.numpy as jnp
import numpy as np
from jax import lax
from jax.experimental import pallas as pl
from jax.experimental.pallas import tpu as pltpu
from jax.experimental.pallas import tpu_sc as plsc

VOCAB = 50257
EMB = 64
NEUR = 1024
K = 50
ROWS_BLK = 256
IDX_CHUNK = 128

_MIN32 = np.int32(-2**31)
_MAX32 = np.int32(2**31 - 1)


def _sc_gather(table, ids2d):
    n_chunks = ids2d.shape[0]
    info = plsc.get_sparse_core_info()
    nc, ns = info.num_cores, info.num_subcores
    nw = nc * ns
    chunks_per_w = n_chunks // nw

    def body(tbl_hbm, idx_hbm, out_hbm, idx_v, row_v, sem):
        wid = lax.axis_index("s") * nc + lax.axis_index("c")
        base = wid * chunks_per_w
        pltpu.sync_copy(idx_hbm.at[pl.ds(base, chunks_per_w)], idx_v)
        for j in range(chunks_per_w):
            pltpu.async_copy(tbl_hbm.at[idx_v.at[j]], row_v, sem).wait()
            pltpu.sync_copy(
                row_v, out_hbm.at[pl.ds((base + j) * IDX_CHUNK, IDX_CHUNK)])

    mesh = plsc.VectorSubcoreMesh(core_axis_name="c", subcore_axis_name="s")
    return pl.kernel(
        body,
        out_type=jax.ShapeDtypeStruct((n_chunks * IDX_CHUNK, EMB), jnp.float32),
        mesh=mesh,
        compiler_params=pltpu.CompilerParams(use_tc_tiling_on_sc=False),
        scratch_types=[
            pltpu.VMEM((chunks_per_w, IDX_CHUNK), jnp.int32),
            pltpu.VMEM((IDX_CHUNK, EMB), jnp.float32),
            pltpu.SemaphoreType.DMA,
        ],
    )(table, ids2d)


def _tc_body(x_ref, w1_ref, b1_ref, w2_ref, b2_ref, o_ref):
    x = x_ref[...]
    h = jnp.dot(x, w1_ref[...],
                preferred_element_type=jnp.float32) + b1_ref[...]
    h = 0.5 * h * (1.0 + lax.erf(h * np.float32(1.0 / np.sqrt(2.0))))
    logits = jnp.dot(h, w2_ref[...],
                     preferred_element_type=jnp.float32) + b2_ref[...]

    bits = lax.bitcast_convert_type(logits, jnp.int32)
    key = jnp.where(bits < 0, bits ^ _MAX32, bits)

    t = jnp.zeros((logits.shape[0], 1), jnp.int32)
    for bit in range(31, -1, -1):
        bv = _MIN32 if bit == 31 else np.int32(1 << bit)
        cand = t | bv
        cnt = jnp.sum((key >= (cand ^ _MIN32)).astype(jnp.int32), axis=1,
                      keepdims=True)
        t = jnp.where(cnt >= K, cand, t)
    thr = t ^ _MIN32
    o_ref[...] = (key >= thr).astype(jnp.float32)


def _tc_fused(emb, W1, b1, W2, b2):
    n_rows = emb.shape[0]
    grid = (n_rows // ROWS_BLK,)
    return pl.pallas_call(
        _tc_body,
        grid=grid,
        in_specs=[
            pl.BlockSpec((ROWS_BLK, EMB), lambda i: (i, 0)),
            pl.BlockSpec((EMB, 2 * NEUR), lambda i: (0, 0)),
            pl.BlockSpec((1, 2 * NEUR), lambda i: (0, 0)),
            pl.BlockSpec((2 * NEUR, NEUR), lambda i: (0, 0)),
            pl.BlockSpec((1, NEUR), lambda i: (0, 0)),
        ],
        out_specs=pl.BlockSpec((ROWS_BLK, NEUR), lambda i: (i, 0)),
        out_shape=jax.ShapeDtypeStruct((n_rows, NEUR), jnp.float32),
    )(emb, W1, b1, W2, b2)


def kernel(token_ids, W_emb, W1, b1, W2, b2):
    B, S = token_ids.shape
    ids2d = token_ids.reshape(-1, IDX_CHUNK).astype(jnp.int32)
    emb = _sc_gather(W_emb, ids2d)
    mask = _tc_fused(emb, W1, b1.reshape(1, -1), W2, b2.reshape(1, -1))
    return mask.reshape(B, S, NEUR)
